# speedup vs baseline: 1.1218x; 1.1218x over previous
import jax
import jax.numpy as jnp
from jax.experimental import pallas as pl

_N = 4096
_D = 128
_BI = 8
_BJ = 128


def _tc_body(adj_ref, x_ref, o_ref):
    def step(c, acc):
        a = adj_ref[:, pl.ds(c * _BJ, _BJ)]
        xs = x_ref[pl.ds(c * _BJ, _BJ), :]
        m = jnp.where(a[:, :, None] > 0, xs[None, :, :], -jnp.inf)
        return jnp.maximum(acc, jnp.max(m, axis=1))

    acc0 = jnp.full((_BI, _D), -jnp.inf, jnp.float32)
    acc = jax.lax.fori_loop(0, _N // _BJ, step, acc0)
    o_ref[...] = jnp.where(acc == -jnp.inf, 0.0, acc)


def kernel(X, adj):
    return pl.pallas_call(
        _tc_body,
        grid=(_N // _BI,),
        in_specs=[
            pl.BlockSpec((_BI, _N), lambda i: (i, 0)),
            pl.BlockSpec((_N, _D), lambda i: (0, 0)),
        ],
        out_specs=pl.BlockSpec((_BI, _D), lambda i: (i, 0)),
        out_shape=jax.ShapeDtypeStruct((_N, _D), jnp.float32),
    )(adj, X)


# speedup vs baseline: 4.5310x; 4.0389x over previous
import functools

import jax
import jax.numpy as jnp
from jax import lax
from jax.experimental import pallas as pl
from jax.experimental.pallas import tpu as pltpu
from jax.experimental.pallas import tpu_sc as plsc

_N = 4096
_D = 128
_L = 16
_NC = 2
_NS = 16
_NW = _NC * _NS
_RPW = _N // _NW
_RG = 8


def _sc_body(x_hbm, adj_hbm, out_hbm, adjb, idxb, rowsb, outb, sem_g):
    cid = lax.axis_index("c")
    sid = lax.axis_index("s")
    wid = sid * _NC + cid
    row0 = wid * _RPW

    iota = lax.iota(jnp.int32, _L)
    zeros_i = jnp.zeros((_L,), jnp.int32)
    neg = jnp.full((_L,), -jnp.inf, jnp.float32)

    def group_fn(t, _):
        r0 = row0 + t * _RG
        pltpu.sync_copy(adj_hbm.at[pl.ds(r0, _RG)], adjb)

        def row_fn(s, _):
            def chunk(c, n):
                v = adjb[s, pl.ds(c * _L, _L)]
                mask = v > 0.0
                cnt = plsc.all_reduce_population_count(mask)[0]

                @pl.when(cnt > 0)
                def _():
                    plsc.store_compressed(
                        idxb.at[pl.ds(n, _L)], c * _L + iota, mask=mask)

                return n + cnt

            nn = lax.fori_loop(0, _N // _L, chunk, jnp.int32(0))

            idx0 = plsc.load_gather(idxb, [zeros_i])
            plsc.store_scatter(idxb, [jnp.full((_L,), nn, jnp.int32) + iota],
                               idx0)

            def gchunk(g, acc):
                idxv = idxb[pl.ds(g * _L, _L)]
                pltpu.async_copy(x_hbm.at[idxv], rowsb, sem_g).wait()
                accl = list(acc)
                for rr in range(_L):
                    for k in range(_D // _L):
                        accl[k] = jnp.maximum(
                            accl[k], rowsb[rr, pl.ds(k * _L, _L)])
                return tuple(accl)

            acc = lax.fori_loop(0, (nn + _L - 1) // _L, gchunk,
                                (neg,) * (_D // _L))
            for k in range(_D // _L):
                outb[s, pl.ds(k * _L, _L)] = jnp.where(
                    acc[k] == -jnp.inf, 0.0, acc[k])
            return 0

        lax.fori_loop(0, _RG, row_fn, 0)
        pltpu.sync_copy(outb, out_hbm.at[pl.ds(r0, _RG)])
        return 0

    lax.fori_loop(0, _RPW // _RG, group_fn, 0)


def _make_sc_kernel():
    mesh = plsc.VectorSubcoreMesh(core_axis_name="c", subcore_axis_name="s",
                                  num_cores=_NC, num_subcores=_NS)
    return functools.partial(
        pl.kernel,
        out_type=jax.ShapeDtypeStruct((_N, _D), jnp.float32),
        mesh=mesh,
        scratch_types=[
            pltpu.VMEM((_RG, _N), jnp.float32),
            pltpu.VMEM((_N + _L,), jnp.int32),
            pltpu.VMEM((_L, _D), jnp.float32),
            pltpu.VMEM((_RG, _D), jnp.float32),
            pltpu.SemaphoreType.DMA,
        ],
        compiler_params=pltpu.CompilerParams(needs_layout_passes=False),
    )(_sc_body)


def kernel(X, adj):
    return _make_sc_kernel()(X, adj)


# speedup vs baseline: 7.8966x; 1.7428x over previous
import functools

import jax
import jax.numpy as jnp
from jax import lax
from jax.experimental import pallas as pl
from jax.experimental.pallas import tpu as pltpu
from jax.experimental.pallas import tpu_sc as plsc

_N = 4096
_D = 128
_L = 16
_NC = 2
_NS = 16
_NW = _NC * _NS
_RPW = _N // _NW
_K = 48
_KC = _K // _L


def _sc_body(x_hbm, adj_hbm, out_hbm, adjb0, adjb1, idxb0, idxb1, rows0,
             rows1, xtrab, outb, sa0, sa1, sg0, sg1, sx):
    cid = lax.axis_index("c")
    sid = lax.axis_index("s")
    wid = sid * _NC + cid
    row0 = wid * _RPW

    adj_bufs = (adjb0, adjb1)
    idx_bufs = (idxb0, idxb1)
    rows_bufs = (rows0, rows1)
    sems_a = (sa0, sa1)
    sems_g = (sg0, sg1)

    iota = lax.iota(jnp.int32, _L)
    ones = jnp.ones((_L,), jnp.int32)
    zeros_i = jnp.zeros((_L,), jnp.int32)
    neg = jnp.full((_L,), -jnp.inf, jnp.float32)

    for q in (0, 1):
        for c in range(_KC):
            idx_bufs[q][pl.ds(c * _L, _L)] = zeros_i

    def fire_adj(t, par):
        pltpu.async_copy(adj_hbm.at[row0 + t], adj_bufs[par], sems_a[par])

    def wait_adj(par):
        pltpu.make_async_copy(adj_hbm.at[0], adj_bufs[par],
                              sems_a[par]).wait()

    def scan_row(par):
        adjq = adj_bufs[par]
        idxq = idx_bufs[par]

        def chunk(c, n_vec):
            v = adjq[pl.ds(c * _L, _L)]
            mask = v > 0.0
            rank = plsc.cumsum(ones, mask=mask)
            plsc.store_scatter(idxq, [n_vec + rank - 1], c * _L + iota,
                               mask=mask)
            return n_vec + plsc.all_reduce_population_count(mask)

        n_vec = lax.fori_loop(0, _N // _L, chunk, zeros_i, unroll=8)
        idx0 = plsc.load_gather(idxq, [zeros_i])
        plsc.store_scatter(idxq, [n_vec + iota], idx0)
        return n_vec[0]

    def fire_gather(par):
        idxq = idx_bufs[par]
        rq = rows_bufs[par]
        for c in range(_KC):
            idxv = idxq[pl.ds(c * _L, _L)]
            pltpu.async_copy(x_hbm.at[idxv], rq.at[pl.ds(c * _L, _L)],
                             sems_g[par])

    def accum_row(r, nn, par, flush):
        rq = rows_bufs[par]
        idxq = idx_bufs[par]
        for c in range(_KC):
            pltpu.make_async_copy(x_hbm.at[pl.ds(0, _L)],
                                  rq.at[pl.ds(c * _L, _L)],
                                  sems_g[par]).wait()

        nnc = jnp.minimum(nn, _K)

        def gchunk(g, acc):
            accl = list(acc)
            for rr in range(_L):
                base = rr
                for k in range(_D // _L):
                    accl[k] = jnp.maximum(
                        accl[k], rq[g * _L + base, pl.ds(k * _L, _L)])
            return tuple(accl)

        acc = lax.fori_loop(0, (nnc + _L - 1) // _L, gchunk,
                            (neg,) * (_D // _L))

        def overflow(acc_in):
            def xchunk(g, acc2):
                idxv = idxq[pl.ds(g * _L, _L)]
                pltpu.async_copy(x_hbm.at[idxv], xtrab, sx).wait()
                accl = list(acc2)
                for rr in range(_L):
                    for k in range(_D // _L):
                        accl[k] = jnp.maximum(
                            accl[k], xtrab[rr, pl.ds(k * _L, _L)])
                return tuple(accl)

            return lax.fori_loop(_KC, (nn + _L - 1) // _L, xchunk, acc_in)

        acc = lax.cond(nn > _K, overflow, lambda a: a, acc)

        s = lax.rem(r, 8)
        for k in range(_D // _L):
            outb[s, pl.ds(k * _L, _L)] = jnp.where(
                acc[k] == -jnp.inf, 0.0, acc[k])

        if flush:
            @pl.when(s == 7)
            def _():
                pltpu.sync_copy(outb, out_hbm.at[pl.ds(row0 + r - 7, 8)])

    def half(r, nn_prev, par, fire_next):
        if fire_next:
            fire_adj(r + 2, par)
        qa = 1 - par
        wait_adj(qa)
        nn_new = scan_row(qa)
        fire_gather(qa)
        accum_row(r, nn_prev, par, flush=(par == 1))
        return nn_new

    fire_adj(0, 0)
    fire_adj(1, 1)
    wait_adj(0)
    nn0 = scan_row(0)
    fire_gather(0)

    def pair_fn(rp, nn_even):
        r = 2 * rp
        nn_odd = half(r, nn_even, 0, True)
        return half(r + 1, nn_odd, 1, True)

    nn0 = lax.fori_loop(0, (_RPW - 2) // 2, pair_fn, nn0)

    nn1 = half(_RPW - 2, nn0, 0, False)
    accum_row(_RPW - 1, nn1, 1, flush=True)


def _make_sc_kernel():
    mesh = plsc.VectorSubcoreMesh(core_axis_name="c", subcore_axis_name="s",
                                  num_cores=_NC, num_subcores=_NS)
    return functools.partial(
        pl.kernel,
        out_type=jax.ShapeDtypeStruct((_N, _D), jnp.float32),
        mesh=mesh,
        scratch_types=[
            pltpu.VMEM((_N,), jnp.float32),
            pltpu.VMEM((_N,), jnp.float32),
            pltpu.VMEM((_N + _L,), jnp.int32),
            pltpu.VMEM((_N + _L,), jnp.int32),
            pltpu.VMEM((_K, _D), jnp.float32),
            pltpu.VMEM((_K, _D), jnp.float32),
            pltpu.VMEM((_L, _D), jnp.float32),
            pltpu.VMEM((8, _D), jnp.float32),
            pltpu.SemaphoreType.DMA,
            pltpu.SemaphoreType.DMA,
            pltpu.SemaphoreType.DMA,
            pltpu.SemaphoreType.DMA,
            pltpu.SemaphoreType.DMA,
        ],
        compiler_params=pltpu.CompilerParams(needs_layout_passes=False,
                                             use_tc_tiling_on_sc=False),
    )(_sc_body)


def kernel(X, adj):
    return _make_sc_kernel()(X, adj)


# speedup vs baseline: 17.3733x; 2.2001x over previous
import functools

import jax
import jax.numpy as jnp
from jax import lax
from jax.experimental import pallas as pl
from jax.experimental.pallas import tpu as pltpu
from jax.experimental.pallas import tpu_sc as plsc

_N = 4096
_D = 128
_W = _N // 16
_L = 16
_NC = 2
_NS = 16
_NW = _NC * _NS
_RPW = _N // _NW
_K = 48
_KC = _K // _L

_BI = 256
_BJ = 1024


def _bits_body(adj_ref, p_ref, o_ref):
    def step(c, acc):
        a = adj_ref[:, pl.ds(c * _BJ, _BJ)].astype(jnp.bfloat16)
        p = p_ref[pl.ds(c * _BJ, _BJ), :]
        return acc + jax.lax.dot_general(
            a, p, (((1,), (0,)), ((), ())),
            preferred_element_type=jnp.float32)

    acc = lax.fori_loop(0, _N // _BJ, step,
                        jnp.zeros((_BI, _W), jnp.float32))
    o_ref[...] = acc.astype(jnp.int32)


def _pack_bits(adj):
    j = lax.broadcasted_iota(jnp.int32, (_N, _W), 0)
    g = lax.broadcasted_iota(jnp.int32, (_N, _W), 1)
    p = jnp.where(j // 16 == g, (1 << (j % 16)), 0).astype(jnp.bfloat16)
    return pl.pallas_call(
        _bits_body,
        grid=(_N // _BI,),
        in_specs=[
            pl.BlockSpec((_BI, _N), lambda i: (i, 0)),
            pl.BlockSpec((_N, _W), lambda i: (0, 0)),
        ],
        out_specs=pl.BlockSpec((_BI, _W), lambda i: (i, 0)),
        out_shape=jax.ShapeDtypeStruct((_N, _W), jnp.int32),
    )(adj, p)


def _sc_body(x_hbm, bits_hbm, out_hbm, adjb0, adjb1, idxb0, idxb1, rows0,
             rows1, xtrab, outb, sa0, sa1, sg0, sg1, sx):
    cid = lax.axis_index("c")
    sid = lax.axis_index("s")
    wid = sid * _NC + cid
    row0 = wid * _RPW

    adj_bufs = (adjb0, adjb1)
    idx_bufs = (idxb0, idxb1)
    rows_bufs = (rows0, rows1)
    sems_a = (sa0, sa1)
    sems_g = (sg0, sg1)

    iota = lax.iota(jnp.int32, _L)
    ones = jnp.ones((_L,), jnp.int32)
    zeros_i = jnp.zeros((_L,), jnp.int32)
    neg = jnp.full((_L,), -jnp.inf, jnp.float32)

    for q in (0, 1):
        for c in range(_KC):
            idx_bufs[q][pl.ds(c * _L, _L)] = zeros_i

    def fire_adj(t, par):
        pltpu.async_copy(bits_hbm.at[row0 + t], adj_bufs[par], sems_a[par])

    def wait_adj(par):
        pltpu.make_async_copy(bits_hbm.at[0], adj_bufs[par],
                              sems_a[par]).wait()

    def scan_row(par):
        wq = adj_bufs[par]
        idxq = idx_bufs[par]

        def group(gg, n_vec):
            w = wq[pl.ds(gg * _L, _L)]

            def cond(st):
                return jnp.any(st[0] != 0)

            def body(st):
                w_, nv = st
                lsb = w_ & (-w_)
                mask_e = w_ != 0
                bit = lax.shift_right_logical(
                    plsc.bitcast(lsb.astype(jnp.float32), jnp.int32), 23
                ) - 127
                jv = gg * (_L * _L) + iota * _L + bit
                rank = plsc.cumsum(ones, mask=mask_e)
                plsc.store_scatter(idxq, [nv + rank - 1], jv, mask=mask_e)
                nv = nv + plsc.all_reduce_population_count(mask_e)
                return (w_ ^ lsb, nv)

            _, n_vec = lax.while_loop(cond, body, (w, n_vec))
            return n_vec

        n_vec = lax.fori_loop(0, _W // _L, group, zeros_i)
        idx0 = plsc.load_gather(idxq, [zeros_i])
        plsc.store_scatter(idxq, [n_vec + iota], idx0)
        return n_vec[0]

    def fire_gather(par):
        idxq = idx_bufs[par]
        rq = rows_bufs[par]
        for c in range(_KC):
            idxv = idxq[pl.ds(c * _L, _L)]
            pltpu.async_copy(x_hbm.at[idxv], rq.at[pl.ds(c * _L, _L)],
                             sems_g[par])

    def accum_row(r, nn, par, flush):
        rq = rows_bufs[par]
        idxq = idx_bufs[par]
        for c in range(_KC):
            pltpu.make_async_copy(x_hbm.at[pl.ds(0, _L)],
                                  rq.at[pl.ds(c * _L, _L)],
                                  sems_g[par]).wait()

        nnc = jnp.minimum(nn, _K)

        def gchunk(g, acc):
            accl = list(acc)
            for rr in range(_L):
                for k in range(_D // _L):
                    accl[k] = jnp.maximum(
                        accl[k], rq[g * _L + rr, pl.ds(k * _L, _L)])
            return tuple(accl)

        acc = lax.fori_loop(0, (nnc + _L - 1) // _L, gchunk,
                            (neg,) * (_D // _L))

        def overflow(acc_in):
            def xchunk(g, acc2):
                idxv = idxq[pl.ds(g * _L, _L)]
                pltpu.async_copy(x_hbm.at[idxv], xtrab, sx).wait()
                accl = list(acc2)
                for rr in range(_L):
                    for k in range(_D // _L):
                        accl[k] = jnp.maximum(
                            accl[k], xtrab[rr, pl.ds(k * _L, _L)])
                return tuple(accl)

            return lax.fori_loop(_KC, (nn + _L - 1) // _L, xchunk, acc_in)

        acc = lax.cond(nn > _K, overflow, lambda a: a, acc)

        s = lax.rem(r, 8)
        for k in range(_D // _L):
            outb[s, pl.ds(k * _L, _L)] = jnp.where(
                acc[k] == -jnp.inf, 0.0, acc[k])

        if flush:
            @pl.when(s == 7)
            def _():
                pltpu.sync_copy(outb, out_hbm.at[pl.ds(row0 + r - 7, 8)])

    def half(r, nn_prev, par, fire_next):
        if fire_next:
            fire_adj(r + 2, par)
        qa = 1 - par
        wait_adj(qa)
        nn_new = scan_row(qa)
        fire_gather(qa)
        accum_row(r, nn_prev, par, flush=(par == 1))
        return nn_new

    fire_adj(0, 0)
    fire_adj(1, 1)
    wait_adj(0)
    nn0 = scan_row(0)
    fire_gather(0)

    def pair_fn(rp, nn_even):
        r = 2 * rp
        nn_odd = half(r, nn_even, 0, True)
        return half(r + 1, nn_odd, 1, True)

    nn0 = lax.fori_loop(0, (_RPW - 2) // 2, pair_fn, nn0)

    nn1 = half(_RPW - 2, nn0, 0, False)
    accum_row(_RPW - 1, nn1, 1, flush=True)


def _make_sc_kernel():
    mesh = plsc.VectorSubcoreMesh(core_axis_name="c", subcore_axis_name="s",
                                  num_cores=_NC, num_subcores=_NS)
    return functools.partial(
        pl.kernel,
        out_type=jax.ShapeDtypeStruct((_N, _D), jnp.float32),
        mesh=mesh,
        scratch_types=[
            pltpu.VMEM((_W,), jnp.int32),
            pltpu.VMEM((_W,), jnp.int32),
            pltpu.VMEM((_N + _L,), jnp.int32),
            pltpu.VMEM((_N + _L,), jnp.int32),
            pltpu.VMEM((_K, _D), jnp.float32),
            pltpu.VMEM((_K, _D), jnp.float32),
            pltpu.VMEM((_L, _D), jnp.float32),
            pltpu.VMEM((8, _D), jnp.float32),
            pltpu.SemaphoreType.DMA,
            pltpu.SemaphoreType.DMA,
            pltpu.SemaphoreType.DMA,
            pltpu.SemaphoreType.DMA,
            pltpu.SemaphoreType.DMA,
        ],
        compiler_params=pltpu.CompilerParams(needs_layout_passes=False,
                                             use_tc_tiling_on_sc=False),
    )(_sc_body)


def kernel(X, adj):
    bits = _pack_bits(adj)
    return _make_sc_kernel()(X, bits)
